# baseline (speedup 1.0000x reference)
import functools

import jax
import jax.numpy as jnp
from jax.experimental import pallas as pl

N_V = 54
N_E = 72
K = 3
DV = 32
DE = 16
L = 5
BBLK = 128


def _sage_block(merged_ref, verts_ref, w1_ref, eb_ref, hw_ref, hb_ref, out_ref):
    merged = merged_ref[...]
    et = jnp.concatenate([merged, merged, merged[:, :6, :]], axis=1)
    vf = verts_ref[...]

    dot = functools.partial(
        jax.lax.dot_general,
        dimension_numbers=(((2,), (0,)), ((), ())),
        preferred_element_type=jnp.float32,
    )

    for i in range(L):
        x = jnp.concatenate([vf, et], axis=2)
        y = dot(x, w1_ref[i])
        p = y[:, :, 0:32]
        pre = None
        for j in range(K):
            s = j + 1
            rolled = jnp.concatenate([p[:, s:, :], p[:, :s, :]], axis=1)
            t = rolled + y[:, :, 32 * (j + 1):32 * (j + 2)]
            pre = t if pre is None else jnp.maximum(pre, t)
        agg = jnp.tanh(pre + eb_ref[i])
        vin = jnp.concatenate([vf, agg], axis=2)
        vf = dot(vin, hw_ref[i]) + hb_ref[i]
        if i < L - 1:
            vf = jnp.tanh(vf)
            nrm = jnp.sum(vf * vf, axis=2, keepdims=True)
            vf = vf * jax.lax.rsqrt(nrm)
    out_ref[...] = vf


def kernel(vertices, edges, eW, eb, hW, hb):
    B = vertices.shape[0]
    merged = edges.reshape(B, N_E // K, K * DE)

    eWv = eW[:, :DV, :]
    eWe = eW[:, DV:, :]
    z = jnp.zeros((L, DE, 32), eW.dtype)
    col0 = jnp.concatenate([eWe, z, z], axis=1)
    col1 = jnp.concatenate([z, eWe, z], axis=1)
    col2 = jnp.concatenate([z, z, eWe], axis=1)
    right = jnp.concatenate([col0, col1, col2], axis=2)
    w1 = jnp.concatenate(
        [
            jnp.concatenate([eWv, jnp.zeros((L, DV, 96), eW.dtype)], axis=2),
            jnp.concatenate([jnp.zeros((L, K * DE, 32), eW.dtype), right], axis=2),
        ],
        axis=1,
    )

    eb3 = eb.reshape(L, 1, 32)
    hb3 = hb.reshape(L, 1, 32)

    grid = (B // BBLK,)
    out = pl.pallas_call(
        _sage_block,
        grid=grid,
        in_specs=[
            pl.BlockSpec((BBLK, N_E // K, K * DE), lambda b: (b, 0, 0)),
            pl.BlockSpec((BBLK, N_V, DV), lambda b: (b, 0, 0)),
            pl.BlockSpec((L, 80, 128), lambda b: (0, 0, 0)),
            pl.BlockSpec((L, 1, 32), lambda b: (0, 0, 0)),
            pl.BlockSpec((L, 64, 32), lambda b: (0, 0, 0)),
            pl.BlockSpec((L, 1, 32), lambda b: (0, 0, 0)),
        ],
        out_specs=pl.BlockSpec((BBLK, N_V, DV), lambda b: (b, 0, 0)),
        out_shape=jax.ShapeDtypeStruct((B, N_V, DV), jnp.float32),
    )(merged, vertices, w1, eb3, hW, hb3)
    return out


# speedup vs baseline: 5.8954x; 5.8954x over previous
import functools

import jax
import jax.numpy as jnp
import numpy as np
from jax.experimental import pallas as pl

N_V = 54
N_E = 72
K = 3
DV = 32
DE = 16
L = 5
P = 4
BB = 128


def _dot(x, w):
    r, c, kf = x.shape
    y = jax.lax.dot_general(
        x.reshape(r * c, kf), w,
        dimension_numbers=(((1,), (0,)), ((), ())),
        preferred_element_type=jnp.float32,
    )
    return y.reshape(r, c, w.shape[1])


def _sage_block(vt_ref, ep_ref, wp_ref, we_ref, wh1_ref, wh2_ref,
                ebp_ref, hbp_ref, ones_ref, out_ref):
    vf = vt_ref[...]
    ep = ep_ref[...]
    ones_bd = ones_ref[...]

    for i in range(L):
        p = _dot(vf, wp_ref[i])
        t = _dot(ep, we_ref[i])
        pre = None
        for j in range(K):
            s = j + 1
            rolled = jnp.concatenate([p[s:], p[:s]], axis=0)
            tj = t[24 * j:24 * (j + 1)]
            tj54 = jnp.concatenate([tj, tj, tj[:6]], axis=0)
            x = rolled + tj54
            pre = x if pre is None else jnp.maximum(pre, x)
        agg = jnp.tanh(pre + ebp_ref[i])
        vf = _dot(vf, wh1_ref[i]) + _dot(agg, wh2_ref[i]) + hbp_ref[i]
        if i < L - 1:
            vf = jnp.tanh(vf)
            ss = _dot(vf * vf, ones_bd)
            vf = vf * jax.lax.rsqrt(ss)
    out_ref[...] = vf


def _kron4(w):
    l, f, o = w.shape
    eye = jnp.eye(P, dtype=w.dtype)
    return (eye[None, :, None, :, None] * w[:, None, :, None, :]).reshape(
        l, P * f, P * o)


def kernel(vertices, edges, eW, eb, hW, hb):
    B = vertices.shape[0]
    B4 = B // P

    vt = vertices.transpose(1, 0, 2).reshape(N_V, B4, P * DV)
    ep = edges.reshape(B4, P, 24, K, DE).transpose(3, 2, 0, 1, 4).reshape(
        K * 24, B4, P * DE)

    eWv = eW[:, :DV, :]
    eWe = eW[:, DV:, :]
    wp = _kron4(eWv)
    we = _kron4(eWe)
    wh1 = _kron4(hW[:, :DV, :])
    wh2 = _kron4(hW[:, DV:, :])
    ebp = jnp.tile(eb, (1, P)).reshape(L, 1, P * 32)
    hbp = jnp.tile(hb, (1, P)).reshape(L, 1, P * 32)
    ones_bd = jnp.kron(jnp.eye(P, dtype=jnp.float32),
                       jnp.ones((DV, DV), jnp.float32))

    grid = (B4 // BB,)
    out = pl.pallas_call(
        _sage_block,
        grid=grid,
        in_specs=[
            pl.BlockSpec((N_V, BB, P * DV), lambda b: (0, b, 0)),
            pl.BlockSpec((K * 24, BB, P * DE), lambda b: (0, b, 0)),
            pl.BlockSpec((L, P * DV, P * DV), lambda b: (0, 0, 0)),
            pl.BlockSpec((L, P * DE, P * DV), lambda b: (0, 0, 0)),
            pl.BlockSpec((L, P * DV, P * DV), lambda b: (0, 0, 0)),
            pl.BlockSpec((L, P * DV, P * DV), lambda b: (0, 0, 0)),
            pl.BlockSpec((L, 1, P * 32), lambda b: (0, 0, 0)),
            pl.BlockSpec((L, 1, P * 32), lambda b: (0, 0, 0)),
            pl.BlockSpec((P * DV, P * DV), lambda b: (0, 0)),
        ],
        out_specs=pl.BlockSpec((N_V, BB, P * DV), lambda b: (0, b, 0)),
        out_shape=jax.ShapeDtypeStruct((N_V, B4, P * DV), jnp.float32),
    )(vt, ep, wp, we, wh1, wh2, ebp, hbp, ones_bd)

    return out.reshape(N_V, B, DV).transpose(1, 0, 2)
